# baseline (speedup 1.0000x reference)
import functools

import jax
import jax.numpy as jnp
from jax import lax
from jax.experimental import pallas as pl
from jax.experimental.pallas import tpu as pltpu
from jax.experimental.pallas import tpu_sc as plsc

_B = 16384
_F = 26
_N = _B * _F
_KEY_RANGE = 64
_NKEYS = 32
_LANES = 16
_NWORKERS = 32
_CHUNK = _N // _NWORKERS
_STEPS = _CHUNK // _LANES
_NOUT = 16
_WPEROUT = _NWORKERS // _NOUT
_OUTLEN = _N // _NOUT


def _lookup_body(x_hbm, dk_hbm, dv_hbm, *refs):
    outs = refs[:_NOUT]
    x_v, out_v, lut_v, dk_v, dv_v = refs[_NOUT:]
    wid = lax.axis_index("s") * 2 + lax.axis_index("c")
    base = wid * _CHUNK
    pltpu.sync_copy(x_hbm.at[pl.ds(base, _CHUNK)], x_v)
    pltpu.sync_copy(dk_hbm, dk_v)
    pltpu.sync_copy(dv_hbm, dv_v)

    zero = jnp.zeros((_LANES,), jnp.int32)
    for i in range(_KEY_RANGE // _LANES):
        lut_v[pl.ds(i * _LANES, _LANES)] = zero
    for j in range(_NKEYS // _LANES):
        k = dk_v[pl.ds(j * _LANES, _LANES)]
        v = dv_v[pl.ds(j * _LANES, _LANES)]
        plsc.store_scatter(lut_v, [k], v)

    def step(s, carry):
        off = s * jnp.int32(_LANES)
        keys = x_v[pl.ds(off, _LANES)]
        out_v[pl.ds(off, _LANES)] = plsc.load_gather(lut_v, [keys])
        return carry

    lax.fori_loop(jnp.int32(0), jnp.int32(_STEPS), step, jnp.int32(0))

    sub = wid % jnp.int32(_WPEROUT)
    off = sub * jnp.int32(_CHUNK)
    grp = wid // jnp.int32(_WPEROUT)
    for j in range(_NOUT):
        @pl.when(grp == jnp.int32(j))
        def _():
            pltpu.sync_copy(out_v, outs[j].at[pl.ds(off, _CHUNK)])


_lookup = functools.partial(
    pl.kernel,
    out_type=tuple(
        jax.ShapeDtypeStruct((_OUTLEN,), jnp.int32) for _ in range(_NOUT)
    ),
    mesh=plsc.VectorSubcoreMesh(core_axis_name="c", subcore_axis_name="s"),
    compiler_params=pltpu.CompilerParams(needs_layout_passes=False),
    scratch_types=[
        pltpu.VMEM((_CHUNK,), jnp.int32),
        pltpu.VMEM((_CHUNK,), jnp.int32),
        pltpu.VMEM((_KEY_RANGE,), jnp.int32),
        pltpu.VMEM((_NKEYS,), jnp.int32),
        pltpu.VMEM((_NKEYS,), jnp.int32),
    ],
)(_lookup_body)


def kernel(input, dict_keys, dict_values):
    x32 = input.reshape(_N).astype(jnp.int32)
    dk32 = dict_keys.astype(jnp.int32)
    dv32 = dict_values.astype(jnp.int32)
    outs = _lookup(x32, dk32, dv32)
    out32 = jnp.concatenate(outs)
    return out32.astype(jnp.int64).reshape(_B, _F)


# speedup vs baseline: 1.0706x; 1.0706x over previous
import functools

import jax
import jax.numpy as jnp
from jax import lax
from jax.experimental import pallas as pl
from jax.experimental.pallas import tpu as pltpu
from jax.experimental.pallas import tpu_sc as plsc

_B = 16384
_F = 26
_N = _B * _F
_KEY_RANGE = 64
_NKEYS = 32
_LANES = 16
_NWORKERS = 32
_CHUNK = _N // _NWORKERS
_NOUT = 8
_WPEROUT = _NWORKERS // _NOUT
_OUTLEN = _N // _NOUT


def _lookup_body(x_hbm, dk_hbm, dv_hbm, *refs):
    outs = refs[:_NOUT]
    x_v, out_v, lut_v, dk_v, dv_v, sem = refs[_NOUT:]
    wid = lax.axis_index("s") * 2 + lax.axis_index("c")
    base = wid * _CHUNK
    in_dma = pltpu.async_copy(x_hbm.at[pl.ds(base, _CHUNK)], x_v, sem)
    pltpu.sync_copy(dk_hbm, dk_v)
    pltpu.sync_copy(dv_hbm, dv_v)

    zero = jnp.zeros((_LANES,), jnp.int32)
    for i in range(_KEY_RANGE // _LANES):
        lut_v[pl.ds(i * _LANES, _LANES)] = zero
    for j in range(_NKEYS // _LANES):
        k = dk_v[pl.ds(j * _LANES, _LANES)]
        v = dv_v[pl.ds(j * _LANES, _LANES)]
        plsc.store_scatter(lut_v, [k], v)
    in_dma.wait()

    @plsc.parallel_loop(
        jnp.int32(0), jnp.int32(_CHUNK), step=jnp.int32(_LANES), unroll=8
    )
    def _step(off):
        keys = x_v[pl.ds(off, _LANES)]
        out_v[pl.ds(off, _LANES)] = plsc.load_gather(lut_v, [keys])

    sub = wid % jnp.int32(_WPEROUT)
    off = sub * jnp.int32(_CHUNK)
    grp = wid // jnp.int32(_WPEROUT)
    for j in range(_NOUT):
        @pl.when(grp == jnp.int32(j))
        def _():
            pltpu.sync_copy(out_v, outs[j].at[pl.ds(off, _CHUNK)])


_lookup = functools.partial(
    pl.kernel,
    out_type=tuple(
        jax.ShapeDtypeStruct((_OUTLEN,), jnp.int32) for _ in range(_NOUT)
    ),
    mesh=plsc.VectorSubcoreMesh(core_axis_name="c", subcore_axis_name="s"),
    compiler_params=pltpu.CompilerParams(needs_layout_passes=False),
    scratch_types=[
        pltpu.VMEM((_CHUNK,), jnp.int32),
        pltpu.VMEM((_CHUNK,), jnp.int32),
        pltpu.VMEM((_KEY_RANGE,), jnp.int32),
        pltpu.VMEM((_NKEYS,), jnp.int32),
        pltpu.VMEM((_NKEYS,), jnp.int32),
        pltpu.SemaphoreType.DMA,
    ],
)(_lookup_body)


def kernel(input, dict_keys, dict_values):
    x32 = input.reshape(_N).astype(jnp.int32)
    dk32 = dict_keys.astype(jnp.int32)
    dv32 = dict_values.astype(jnp.int32)
    outs = _lookup(x32, dk32, dv32)
    out32 = jnp.concatenate(outs)
    return out32.astype(jnp.int64).reshape(_B, _F)
